# baseline (speedup 1.0000x reference)
import functools

import jax
import jax.numpy as jnp
from jax import lax
from jax.experimental import pallas as pl
from jax.experimental.pallas import tpu as pltpu
from jax.experimental.pallas import tpu_sc as plsc

_NC = 2
_NS = 16
_NW = _NC * _NS
_CHUNK = 128
_BLK = 32768
_LANES = 16


def _tc_project_table(tableT, W, b):
    d, v = tableT.shape
    o = W.shape[0]
    grid = (v + _BLK - 1) // _BLK

    def body(t_ref, w_ref, b_ref, *o_refs):
        res = lax.dot_general(
            w_ref[...], t_ref[...], (((1,), (0,)), ((), ())),
            preferred_element_type=jnp.float32,
        )
        for jj in range(o // 2):
            lo = (res[2 * jj, :] + b_ref[2 * jj]).astype(jnp.bfloat16)
            hi = (res[2 * jj + 1, :] + b_ref[2 * jj + 1]).astype(jnp.bfloat16)
            lo32 = lax.bitcast_convert_type(lo, jnp.uint16).astype(jnp.uint32)
            hi32 = lax.bitcast_convert_type(hi, jnp.uint16).astype(jnp.uint32)
            o_refs[jj][...] = (hi32 << 16) | lo32

    return pl.pallas_call(
        body,
        grid=(grid,),
        in_specs=[
            pl.BlockSpec((d, _BLK), lambda i: (0, i)),
            pl.BlockSpec((o, d), lambda i: (0, 0)),
            pl.BlockSpec(memory_space=pltpu.SMEM),
        ],
        out_specs=tuple(
            pl.BlockSpec((_BLK,), lambda i: (i,)) for _ in range(o // 2)
        ),
        out_shape=tuple(
            jax.ShapeDtypeStruct((v,), jnp.uint32) for _ in range(o // 2)
        ),
    )(tableT, W, b)


def _sc_gather_unpack(packed, idx, o):
    op = len(packed)
    batch = idx.shape[0]
    b_per_w = batch // _NW
    n_chunks = b_per_w // _CHUNK

    mesh = plsc.VectorSubcoreMesh(core_axis_name="c", subcore_axis_name="s")

    @functools.partial(
        pl.kernel,
        mesh=mesh,
        out_type=jax.ShapeDtypeStruct((o, batch), jnp.float32),
        scratch_types=(
            [pltpu.VMEM((b_per_w,), jnp.int32)]
            + [pltpu.VMEM((b_per_w,), jnp.uint32) for _ in range(op)]
            + [
                pltpu.VMEM((1, o * b_per_w), jnp.float32),
                pltpu.SemaphoreType.DMA,
            ]
        ),
    )
    def gather(*refs):
        col_hbms = refs[:op]
        idx_hbm = refs[op]
        out_hbm = refs[op + 1]
        idx_v = refs[op + 2]
        vals_v = refs[op + 3 : op + 3 + op]
        unp_v, sem = refs[op + 3 + op :]
        wid = lax.axis_index("s") * _NC + lax.axis_index("c")
        pltpu.sync_copy(idx_hbm.at[pl.ds(wid * b_per_w, b_per_w)], idx_v)
        copies = []
        for jj in range(op):
            for i in range(n_chunks):
                copies.append(
                    pltpu.async_copy(
                        col_hbms[jj].at[idx_v.at[pl.ds(i * _CHUNK, _CHUNK)]],
                        vals_v[jj].at[pl.ds(i * _CHUNK, _CHUNK)],
                        sem,
                    )
                )
        for cp in copies:
            cp.wait()
        himask = jnp.uint32(0xFFFF0000)
        shamt = jnp.uint32(16)
        for jj in range(op):
            for i in range(b_per_w // _LANES):
                w = vals_v[jj][pl.ds(i * _LANES, _LANES)]
                unp_v[0, pl.ds(2 * jj * b_per_w + i * _LANES, _LANES)] = (
                    lax.bitcast_convert_type(w << shamt, jnp.float32)
                )
                unp_v[
                    0, pl.ds((2 * jj + 1) * b_per_w + i * _LANES, _LANES)
                ] = lax.bitcast_convert_type(w & himask, jnp.float32)
        for r in range(o):
            pltpu.sync_copy(
                unp_v.at[pl.ds(0, 1), pl.ds(r * b_per_w, b_per_w)],
                out_hbm.at[pl.ds(r, 1), pl.ds(wid * b_per_w, b_per_w)],
            )

    return gather(*packed, idx)


def kernel(x_id, table, W, b):
    tableT = table.T
    packed = _tc_project_table(tableT, W, b)
    pout = _sc_gather_unpack(packed, x_id.astype(jnp.int32), W.shape[0])
    return pout.T


# speedup vs baseline: 1.0552x; 1.0552x over previous
import functools

import jax
import jax.numpy as jnp
from jax import lax
from jax.experimental import pallas as pl
from jax.experimental.pallas import tpu as pltpu
from jax.experimental.pallas import tpu_sc as plsc

_NC = 2
_NS = 16
_NW = _NC * _NS
_CHUNK = 128
_BLK = 32768


def _tc_project_table(tableT, W, b):
    d, v = tableT.shape
    o = W.shape[0]
    grid = (v + _BLK - 1) // _BLK

    def body(t_ref, w_ref, b_ref, *o_refs):
        res = lax.dot_general(
            w_ref[...], t_ref[...], (((1,), (0,)), ((), ())),
            preferred_element_type=jnp.float32,
        )
        for j in range(o):
            o_refs[j][...] = res[j, :] + b_ref[j]

    return pl.pallas_call(
        body,
        grid=(grid,),
        in_specs=[
            pl.BlockSpec((d, _BLK), lambda i: (0, i)),
            pl.BlockSpec((o, d), lambda i: (0, 0)),
            pl.BlockSpec(memory_space=pltpu.SMEM),
        ],
        out_specs=tuple(pl.BlockSpec((_BLK,), lambda i: (i,)) for _ in range(o)),
        out_shape=tuple(
            jax.ShapeDtypeStruct((v,), jnp.float32) for _ in range(o)
        ),
    )(tableT, W, b)


def _sc_gather_cols(cols, idx):
    o = len(cols)
    batch = idx.shape[0]
    b_per_w = batch // _NW
    n_chunks = b_per_w // _CHUNK

    mesh = plsc.VectorSubcoreMesh(core_axis_name="c", subcore_axis_name="s")

    @functools.partial(
        pl.kernel,
        mesh=mesh,
        out_type=jax.ShapeDtypeStruct((o, batch), jnp.float32),
        scratch_types=[
            pltpu.VMEM((b_per_w,), jnp.int32),
            pltpu.VMEM((o, b_per_w), jnp.float32),
            pltpu.SemaphoreType.DMA,
        ],
    )
    def gather(*refs):
        col_hbms = refs[:o]
        idx_hbm = refs[o]
        out_hbm = refs[o + 1]
        idx_v, vals_v, sem = refs[o + 2], refs[o + 3], refs[o + 4]
        wid = lax.axis_index("s") * _NC + lax.axis_index("c")
        pltpu.sync_copy(idx_hbm.at[pl.ds(wid * b_per_w, b_per_w)], idx_v)
        copies = []
        for j in range(o):
            for i in range(n_chunks):
                copies.append(
                    pltpu.async_copy(
                        col_hbms[j].at[idx_v.at[pl.ds(i * _CHUNK, _CHUNK)]],
                        vals_v.at[j, pl.ds(i * _CHUNK, _CHUNK)],
                        sem,
                    )
                )
        for cp in copies:
            cp.wait()
        pltpu.sync_copy(
            vals_v, out_hbm.at[:, pl.ds(wid * b_per_w, b_per_w)]
        )

    return gather(*cols, idx)


def kernel(x_id, table, W, b):
    tableT = table.T
    cols = _tc_project_table(tableT, W, b)
    pout = _sc_gather_cols(cols, x_id.astype(jnp.int32))
    return pout.T
